# Initial kernel scaffold; baseline (speedup 1.0000x reference)
import jax
import jax.numpy as jnp
from jax.experimental import pallas as pl


def kernel(X, history_item_id, history_item_value, history_user_id, history_user_value, W_user, W_item, W_ufc, b_ufc, W_ifc, b_ifc):
    raise NotImplementedError("write your pallas kernel here")


# speedup vs baseline: 16.7826x; 16.7826x over previous
import jax
import jax.numpy as jnp
from jax import lax
from jax.experimental import pallas as pl
from jax.experimental.pallas import tpu as pltpu
from jax.experimental.pallas import tpu_sc as plsc

B = 1024
H = 50
HP = 64
EMB = 64
NV = 100000
NC = 2
NS = 16
NW = NC * NS
BPW = B // NW
HALF = BPW // 2
GRP = 8
_sc_params = pltpu.CompilerParams(use_tc_tiling_on_sc=False)


def _sc_body(uid_hbm, iid_hbm, hii_hbm, hiv_hbm, hui_hbm, huv_hbm,
             w2_hbm,
             ids_u_out, vals_u_out, rows_u_out,
             ids_i_out, vals_i_out, rows_i_out,
             bidx_v, ids_v, vals_v, rows_v, sem):
  wid = lax.axis_index("s") * NC + lax.axis_index("c")
  base = wid * BPW

  def one_side(id_src, hid_hbm, hval_hbm, w_hbm, ids_out, vals_out, rows_out):
    pltpu.sync_copy(id_src.at[pl.ds(base, BPW)], bidx_v)
    pltpu.async_copy(hid_hbm.at[bidx_v], ids_v, sem).wait()
    pltpu.async_copy(hval_hbm.at[bidx_v], vals_v, sem).wait()
    pltpu.sync_copy(ids_v, ids_out.at[pl.ds(base, BPW)])
    pltpu.sync_copy(vals_v, vals_out.at[pl.ds(base, BPW)])
    for half in range(2):
      for g in range(HALF // GRP):
        k0 = half * HALF + g * GRP
        cps = [pltpu.async_copy(
            w_hbm.at[ids_v.at[k0 + j]],
            rows_v.at[g * GRP + j], sem)
            for j in range(GRP)]
        for cp in cps:
          cp.wait()
      pltpu.sync_copy(rows_v,
                      rows_out.at[pl.ds(base + half * HALF, HALF)])

  one_side(uid_hbm, hii_hbm, hiv_hbm, w2_hbm, ids_u_out, vals_u_out,
           rows_u_out)
  one_side(iid_hbm, hui_hbm, huv_hbm, w2_hbm, ids_i_out, vals_i_out,
           rows_i_out)


def _sc_gather(user_ids, item_ids, hii, hiv, hui, huv, w2):
  mesh = plsc.VectorSubcoreMesh(core_axis_name="c", subcore_axis_name="s")
  out_type = (
      jax.ShapeDtypeStruct((B, HP), jnp.int32),
      jax.ShapeDtypeStruct((B, HP), jnp.float32),
      jax.ShapeDtypeStruct((B, HP, EMB), jnp.float32),
      jax.ShapeDtypeStruct((B, HP), jnp.int32),
      jax.ShapeDtypeStruct((B, HP), jnp.float32),
      jax.ShapeDtypeStruct((B, HP, EMB), jnp.float32),
  )
  scratch = [
      pltpu.VMEM((BPW,), jnp.int32),
      pltpu.VMEM((BPW, HP), jnp.int32),
      pltpu.VMEM((BPW, HP), jnp.float32),
      pltpu.VMEM((HALF, HP, EMB), jnp.float32),
      pltpu.SemaphoreType.DMA,
  ]
  fn = pl.kernel(_sc_body, out_type=out_type, mesh=mesh,
                 scratch_types=scratch, compiler_params=_sc_params)
  return fn(user_ids, item_ids, hii, hiv, hui, huv, w2)


BB = 128


def _tc_body(ids_u_ref, vals_u_ref, rows_u_ref, ids_i_ref, vals_i_ref,
             rows_i_ref, wufc_ref, bufc_ref, wifc_ref, bifc_ref, out_ref):
  def side_emb(ids, vals, rows):
    dup = jnp.zeros(ids.shape, jnp.bool_)
    col = lax.broadcasted_iota(jnp.int32, ids.shape, 1)
    for hp in range(1, H):
      dup = dup | ((ids == ids[:, hp:hp + 1]) & (col < hp))
    veff = jnp.where(dup, 0.0, vals)
    return jnp.sum(veff[:, :, None] * rows, axis=1)

  emb_u = side_emb(ids_u_ref[...], vals_u_ref[...], rows_u_ref[...])
  emb_i = side_emb(ids_i_ref[...], vals_i_ref[...], rows_i_ref[...])
  user = jnp.dot(emb_u, wufc_ref[...],
                 preferred_element_type=jnp.float32) + bufc_ref[...]
  item = jnp.dot(emb_i, wifc_ref[...],
                 preferred_element_type=jnp.float32) + bifc_ref[...]
  s = jnp.sum(user * item, axis=1)
  out_ref[0, :] = jax.nn.sigmoid(s)


def _tc_dense(ids_u, vals_u, rows_u, ids_i, vals_i, rows_i,
              w_ufc, b_ufc, w_ifc, b_ifc):
  grid = (B // BB,)
  bhp = pl.BlockSpec((BB, HP), lambda i: (i, 0))
  bhe = pl.BlockSpec((BB, HP, EMB), lambda i: (i, 0, 0))
  full = pl.BlockSpec((EMB, EMB), lambda i: (0, 0))
  bias = pl.BlockSpec((1, EMB), lambda i: (0, 0))
  out = pl.pallas_call(
      _tc_body,
      grid=grid,
      in_specs=[bhp, bhp, bhe, bhp, bhp, bhe, full, bias, full, bias],
      out_specs=pl.BlockSpec((1, BB), lambda i: (0, i)),
      out_shape=jax.ShapeDtypeStruct((1, B), jnp.float32),
  )(ids_u, vals_u, rows_u, ids_i, vals_i, rows_i,
    w_ufc, b_ufc.reshape(1, EMB), w_ifc, b_ifc.reshape(1, EMB))
  return out.reshape(B)


def _pad_val_table(t):
  return jnp.concatenate([t, jnp.zeros((NV, HP - H), jnp.float32)], axis=1)


def _pad_id_table(t, base):
  r = jnp.arange(NV, dtype=jnp.int32)[:, None]
  c = jnp.arange(HP - H, dtype=jnp.int32)[None, :]
  return jnp.concatenate(
      [t.astype(jnp.int32) + base, (r * 7 + c * 131 + 1) % NV + base],
      axis=1)


@jax.jit
def kernel(X, history_item_id, history_item_value, history_user_id,
           history_user_value, W_user, W_item, W_ufc, b_ufc, W_ifc, b_ifc):
  user_ids = X[:, 0].astype(jnp.int32)
  item_ids = X[:, 1].astype(jnp.int32)
  w2 = jnp.concatenate([W_user, W_item], axis=0)
  ids_u, vals_u, rows_u, ids_i, vals_i, rows_i = _sc_gather(
      user_ids, item_ids,
      _pad_id_table(history_item_id, 0), _pad_val_table(history_item_value),
      _pad_id_table(history_user_id, NV), _pad_val_table(history_user_value),
      w2)
  return _tc_dense(ids_u, vals_u, rows_u, ids_i, vals_i, rows_i,
                   W_ufc, b_ufc, W_ifc, b_ifc)


# speedup vs baseline: 19.7106x; 1.1745x over previous
import jax
import jax.numpy as jnp
from jax import lax
from jax.experimental import pallas as pl
from jax.experimental.pallas import tpu as pltpu
from jax.experimental.pallas import tpu_sc as plsc

B = 1024
H = 50
HP = 64
EMB = 64
NV = 100000
NC = 2
NS = 16
NW = NC * NS
BPW = B // NW
HALF = BPW // 2
GRP = 8
_sc_params = pltpu.CompilerParams(use_tc_tiling_on_sc=False)


def _sc_body(uid_hbm, iid_hbm, hii_hbm, hiv_hbm, hui_hbm, huv_hbm,
             wu_hbm, wi_hbm,
             ids_u_out, vals_u_out, rows_u_out,
             ids_i_out, vals_i_out, rows_i_out,
             bidx_v, ids_v, vals_v, rows_v, sem):
  wid = lax.axis_index("s") * NC + lax.axis_index("c")
  base = wid * BPW

  def one_side(id_src, hid_hbm, hval_hbm, w_hbm, ids_out, vals_out, rows_out):
    pltpu.sync_copy(id_src.at[pl.ds(base, BPW)], bidx_v)
    pltpu.async_copy(hid_hbm.at[bidx_v], ids_v, sem).wait()
    pltpu.async_copy(hval_hbm.at[bidx_v], vals_v, sem).wait()
    pltpu.sync_copy(ids_v, ids_out.at[pl.ds(base, BPW)])
    pltpu.sync_copy(vals_v, vals_out.at[pl.ds(base, BPW)])
    for half in range(2):
      for g in range(HALF // GRP):
        k0 = half * HALF + g * GRP
        cps = [pltpu.async_copy(
            w_hbm.at[ids_v.at[k0 + j]],
            rows_v.at[g * GRP + j], sem)
            for j in range(GRP)]
        for cp in cps:
          cp.wait()
      pltpu.sync_copy(rows_v,
                      rows_out.at[pl.ds(base + half * HALF, HALF)])

  one_side(uid_hbm, hii_hbm, hiv_hbm, wu_hbm, ids_u_out, vals_u_out,
           rows_u_out)
  one_side(iid_hbm, hui_hbm, huv_hbm, wi_hbm, ids_i_out, vals_i_out,
           rows_i_out)


def _sc_gather(user_ids, item_ids, hii, hiv, hui, huv, wu, wi):
  mesh = plsc.VectorSubcoreMesh(core_axis_name="c", subcore_axis_name="s")
  out_type = (
      jax.ShapeDtypeStruct((B, HP), jnp.int32),
      jax.ShapeDtypeStruct((B, HP), jnp.float32),
      jax.ShapeDtypeStruct((B, HP, EMB), jnp.float32),
      jax.ShapeDtypeStruct((B, HP), jnp.int32),
      jax.ShapeDtypeStruct((B, HP), jnp.float32),
      jax.ShapeDtypeStruct((B, HP, EMB), jnp.float32),
  )
  scratch = [
      pltpu.VMEM((BPW,), jnp.int32),
      pltpu.VMEM((BPW, HP), jnp.int32),
      pltpu.VMEM((BPW, HP), jnp.float32),
      pltpu.VMEM((HALF, HP, EMB), jnp.float32),
      pltpu.SemaphoreType.DMA,
  ]
  fn = pl.kernel(_sc_body, out_type=out_type, mesh=mesh,
                 scratch_types=scratch, compiler_params=_sc_params)
  return fn(user_ids, item_ids, hii, hiv, hui, huv, wu, wi)


BB = 128


def _tc_body(ids_u_ref, vals_u_ref, rows_u_ref, ids_i_ref, vals_i_ref,
             rows_i_ref, wufc_ref, bufc_ref, wifc_ref, bifc_ref, out_ref):
  def side_emb(ids, vals, rows):
    dup = jnp.zeros(ids.shape, jnp.bool_)
    col = lax.broadcasted_iota(jnp.int32, ids.shape, 1)
    for hp in range(1, H):
      dup = dup | ((ids == ids[:, hp:hp + 1]) & (col < hp))
    veff = jnp.where(dup, 0.0, vals)
    return jnp.sum(veff[:, :, None] * rows, axis=1)

  emb_u = side_emb(ids_u_ref[...], vals_u_ref[...], rows_u_ref[...])
  emb_i = side_emb(ids_i_ref[...], vals_i_ref[...], rows_i_ref[...])
  user = jnp.dot(emb_u, wufc_ref[...],
                 preferred_element_type=jnp.float32) + bufc_ref[...]
  item = jnp.dot(emb_i, wifc_ref[...],
                 preferred_element_type=jnp.float32) + bifc_ref[...]
  s = jnp.sum(user * item, axis=1)
  out_ref[0, :] = jax.nn.sigmoid(s)


def _tc_dense(ids_u, vals_u, rows_u, ids_i, vals_i, rows_i,
              w_ufc, b_ufc, w_ifc, b_ifc):
  grid = (B // BB,)
  bhp = pl.BlockSpec((BB, HP), lambda i: (i, 0))
  bhe = pl.BlockSpec((BB, HP, EMB), lambda i: (i, 0, 0))
  full = pl.BlockSpec((EMB, EMB), lambda i: (0, 0))
  bias = pl.BlockSpec((1, EMB), lambda i: (0, 0))
  out = pl.pallas_call(
      _tc_body,
      grid=grid,
      in_specs=[bhp, bhp, bhe, bhp, bhp, bhe, full, bias, full, bias],
      out_specs=pl.BlockSpec((1, BB), lambda i: (0, i)),
      out_shape=jax.ShapeDtypeStruct((1, B), jnp.float32),
  )(ids_u, vals_u, rows_u, ids_i, vals_i, rows_i,
    w_ufc, b_ufc.reshape(1, EMB), w_ifc, b_ifc.reshape(1, EMB))
  return out.reshape(B)


def _pad_val_table(t):
  return jnp.concatenate([t, jnp.zeros((NV, HP - H), jnp.float32)], axis=1)


def _pad_id_table(t, base):
  r = jnp.arange(NV, dtype=jnp.int32)[:, None]
  c = jnp.arange(HP - H, dtype=jnp.int32)[None, :]
  return jnp.concatenate(
      [t.astype(jnp.int32) + base, (r * 7 + c * 131 + 1) % NV + base],
      axis=1)


@jax.jit
def kernel(X, history_item_id, history_item_value, history_user_id,
           history_user_value, W_user, W_item, W_ufc, b_ufc, W_ifc, b_ifc):
  user_ids = X[:, 0].astype(jnp.int32)
  item_ids = X[:, 1].astype(jnp.int32)
  ids_u, vals_u, rows_u, ids_i, vals_i, rows_i = _sc_gather(
      user_ids, item_ids,
      _pad_id_table(history_item_id, 0), _pad_val_table(history_item_value),
      _pad_id_table(history_user_id, 0), _pad_val_table(history_user_value),
      W_user, W_item)
  return _tc_dense(ids_u, vals_u, rows_u, ids_i, vals_i, rows_i,
                   W_ufc, b_ufc, W_ifc, b_ifc)


# speedup vs baseline: 20.2695x; 1.0284x over previous
import jax
import jax.numpy as jnp
from jax import lax
from jax.experimental import pallas as pl
from jax.experimental.pallas import tpu as pltpu
from jax.experimental.pallas import tpu_sc as plsc

B = 1024
H = 50
HP = 56
EMB = 64
NV = 100000
NC = 2
NS = 16
NW = NC * NS
BPW = B // NW
HALF = BPW // 2
GRP = 8
_sc_params = pltpu.CompilerParams(use_tc_tiling_on_sc=False)


def _sc_body(uid_hbm, iid_hbm, hii_hbm, hiv_hbm, hui_hbm, huv_hbm,
             wu_hbm, wi_hbm,
             ids_u_out, vals_u_out, rows_u_out,
             ids_i_out, vals_i_out, rows_i_out,
             bidx_v, ids_v, vals_v, rows_v, sem):
  wid = lax.axis_index("s") * NC + lax.axis_index("c")
  base = wid * BPW

  def one_side(id_src, hid_hbm, hval_hbm, w_hbm, ids_out, vals_out, rows_out):
    pltpu.sync_copy(id_src.at[pl.ds(base, BPW)], bidx_v)
    pltpu.async_copy(hid_hbm.at[bidx_v], ids_v, sem).wait()
    pltpu.async_copy(hval_hbm.at[bidx_v], vals_v, sem).wait()
    pltpu.sync_copy(ids_v, ids_out.at[pl.ds(base, BPW)])
    pltpu.sync_copy(vals_v, vals_out.at[pl.ds(base, BPW)])
    for half in range(2):
      for g in range(HALF // GRP):
        k0 = half * HALF + g * GRP
        cps = [pltpu.async_copy(
            w_hbm.at[ids_v.at[k0 + j]],
            rows_v.at[g * GRP + j], sem)
            for j in range(GRP)]
        for cp in cps:
          cp.wait()
      pltpu.sync_copy(rows_v,
                      rows_out.at[pl.ds(base + half * HALF, HALF)])

  one_side(uid_hbm, hii_hbm, hiv_hbm, wu_hbm, ids_u_out, vals_u_out,
           rows_u_out)
  one_side(iid_hbm, hui_hbm, huv_hbm, wi_hbm, ids_i_out, vals_i_out,
           rows_i_out)


def _sc_gather(user_ids, item_ids, hii, hiv, hui, huv, wu, wi):
  mesh = plsc.VectorSubcoreMesh(core_axis_name="c", subcore_axis_name="s")
  out_type = (
      jax.ShapeDtypeStruct((B, HP), jnp.int32),
      jax.ShapeDtypeStruct((B, HP), jnp.float32),
      jax.ShapeDtypeStruct((B, HP, EMB), jnp.float32),
      jax.ShapeDtypeStruct((B, HP), jnp.int32),
      jax.ShapeDtypeStruct((B, HP), jnp.float32),
      jax.ShapeDtypeStruct((B, HP, EMB), jnp.float32),
  )
  scratch = [
      pltpu.VMEM((BPW,), jnp.int32),
      pltpu.VMEM((BPW, HP), jnp.int32),
      pltpu.VMEM((BPW, HP), jnp.float32),
      pltpu.VMEM((HALF, HP, EMB), jnp.float32),
      pltpu.SemaphoreType.DMA,
  ]
  fn = pl.kernel(_sc_body, out_type=out_type, mesh=mesh,
                 scratch_types=scratch, compiler_params=_sc_params)
  return fn(user_ids, item_ids, hii, hiv, hui, huv, wu, wi)


BB = 128


def _tc_body(ids_u_ref, vals_u_ref, rows_u_ref, ids_i_ref, vals_i_ref,
             rows_i_ref, wufc_ref, bufc_ref, wifc_ref, bifc_ref, out_ref):
  def side_emb(ids, vals, rows):
    dup = jnp.zeros(ids.shape, jnp.bool_)
    col = lax.broadcasted_iota(jnp.int32, ids.shape, 1)
    for hp in range(1, H):
      dup = dup | ((ids == ids[:, hp:hp + 1]) & (col < hp))
    veff = jnp.where(dup, 0.0, vals)
    return jnp.sum(veff[:, :, None] * rows, axis=1)

  emb_u = side_emb(ids_u_ref[...], vals_u_ref[...], rows_u_ref[...])
  emb_i = side_emb(ids_i_ref[...], vals_i_ref[...], rows_i_ref[...])
  user = jnp.dot(emb_u, wufc_ref[...],
                 preferred_element_type=jnp.float32) + bufc_ref[...]
  item = jnp.dot(emb_i, wifc_ref[...],
                 preferred_element_type=jnp.float32) + bifc_ref[...]
  s = jnp.sum(user * item, axis=1)
  out_ref[0, :] = jax.nn.sigmoid(s)


def _tc_dense(ids_u, vals_u, rows_u, ids_i, vals_i, rows_i,
              w_ufc, b_ufc, w_ifc, b_ifc):
  grid = (B // BB,)
  bhp = pl.BlockSpec((BB, HP), lambda i: (i, 0))
  bhe = pl.BlockSpec((BB, HP, EMB), lambda i: (i, 0, 0))
  full = pl.BlockSpec((EMB, EMB), lambda i: (0, 0))
  bias = pl.BlockSpec((1, EMB), lambda i: (0, 0))
  out = pl.pallas_call(
      _tc_body,
      grid=grid,
      in_specs=[bhp, bhp, bhe, bhp, bhp, bhe, full, bias, full, bias],
      out_specs=pl.BlockSpec((1, BB), lambda i: (0, i)),
      out_shape=jax.ShapeDtypeStruct((1, B), jnp.float32),
  )(ids_u, vals_u, rows_u, ids_i, vals_i, rows_i,
    w_ufc, b_ufc.reshape(1, EMB), w_ifc, b_ifc.reshape(1, EMB))
  return out.reshape(B)


def _pad_val_table(t):
  return jnp.concatenate([t, jnp.zeros((NV, HP - H), jnp.float32)], axis=1)


def _pad_id_table(t, base):
  r = jnp.arange(NV, dtype=jnp.int32)[:, None]
  c = jnp.arange(HP - H, dtype=jnp.int32)[None, :]
  return jnp.concatenate(
      [t.astype(jnp.int32) + base, (r * 7 + c * 131 + 1) % NV + base],
      axis=1)


@jax.jit
def kernel(X, history_item_id, history_item_value, history_user_id,
           history_user_value, W_user, W_item, W_ufc, b_ufc, W_ifc, b_ifc):
  user_ids = X[:, 0].astype(jnp.int32)
  item_ids = X[:, 1].astype(jnp.int32)
  ids_u, vals_u, rows_u, ids_i, vals_i, rows_i = _sc_gather(
      user_ids, item_ids,
      _pad_id_table(history_item_id, 0), _pad_val_table(history_item_value),
      _pad_id_table(history_user_id, 0), _pad_val_table(history_user_value),
      W_user, W_item)
  return _tc_dense(ids_u, vals_u, rows_u, ids_i, vals_i, rows_i,
                   W_ufc, b_ufc, W_ifc, b_ifc)
